# Initial kernel scaffold; baseline (speedup 1.0000x reference)
import jax
import jax.numpy as jnp
from jax.experimental import pallas as pl


def kernel(input, emb_donor, emb_project, W1, b1, W2, b2):
    raise NotImplementedError("write your pallas kernel here")


# speedup vs baseline: 1.1629x; 1.1629x over previous
import functools

import jax
import jax.numpy as jnp
from jax import lax
from jax.experimental import pallas as pl
from jax.experimental.pallas import tpu as pltpu
from jax.experimental.pallas import tpu_sc as plsc

_B = 16384
_D = 64
_LIN = 256
_NC, _NS = 2, 16
_NW = _NC * _NS
_BPW = _B // _NW
_CH = 128
_NCH = _BPW // _CH

_sc_mesh = plsc.VectorSubcoreMesh(core_axis_name="c", subcore_axis_name="s")


@functools.partial(
    pl.kernel,
    out_type=[
        jax.ShapeDtypeStruct((_B, _D), jnp.float32),
        jax.ShapeDtypeStruct((_B, _D), jnp.float32),
    ],
    mesh=_sc_mesh,
    compiler_params=pltpu.CompilerParams(use_tc_tiling_on_sc=False),
    scratch_types=[
        pltpu.VMEM((_BPW,), jnp.int32),
        pltpu.VMEM((_BPW,), jnp.int32),
        pltpu.VMEM((_BPW, _D), jnp.float32),
        pltpu.VMEM((_BPW, _D), jnp.float32),
        pltpu.SemaphoreType.DMA,
    ],
)
def _sc_gather(didx_hbm, pidx_hbm, donor_tbl, proj_tbl, outd, outp,
               idx_d, idx_p, rows_d, rows_p, sem):
    wid = lax.axis_index("s") * _NC + lax.axis_index("c")
    base = wid * _BPW
    pltpu.sync_copy(didx_hbm.at[pl.ds(base, _BPW)], idx_d)
    pltpu.sync_copy(pidx_hbm.at[pl.ds(base, _BPW)], idx_p)
    copies = []
    for j in range(_NCH):
        sl = pl.ds(j * _CH, _CH)
        copies.append(pltpu.async_copy(
            donor_tbl.at[idx_d.at[sl]], rows_d.at[sl], sem))
        copies.append(pltpu.async_copy(
            proj_tbl.at[idx_p.at[sl]], rows_p.at[sl], sem))
    for cp in copies:
        cp.wait()
    pltpu.sync_copy(rows_d, outd.at[pl.ds(base, _BPW)])
    pltpu.sync_copy(rows_p, outp.at[pl.ds(base, _BPW)])


_BLK = 2048


def _mlp_body(d_ref, p_ref, w1a_ref, w1b_ref, b1_ref, w2_ref, b2_ref, o_ref):
    h = jnp.dot(d_ref[...], w1a_ref[...], preferred_element_type=jnp.float32)
    h = h + jnp.dot(p_ref[...], w1b_ref[...], preferred_element_type=jnp.float32)
    h = jnp.maximum(h + b1_ref[...], 0.0)
    o_ref[...] = jnp.dot(h, w2_ref[...], preferred_element_type=jnp.float32) + b2_ref[...]


_mlp = pl.pallas_call(
    _mlp_body,
    grid=(_B // _BLK,),
    in_specs=[
        pl.BlockSpec((_BLK, _D), lambda i: (i, 0)),
        pl.BlockSpec((_BLK, _D), lambda i: (i, 0)),
        pl.BlockSpec((_D, _LIN), lambda i: (0, 0)),
        pl.BlockSpec((_D, _LIN), lambda i: (0, 0)),
        pl.BlockSpec((1, _LIN), lambda i: (0, 0)),
        pl.BlockSpec((_LIN, 1), lambda i: (0, 0)),
        pl.BlockSpec((1, 1), lambda i: (0, 0)),
    ],
    out_specs=pl.BlockSpec((_BLK, 1), lambda i: (i, 0)),
    out_shape=jax.ShapeDtypeStruct((_B, 1), jnp.float32),
)


@jax.jit
def kernel(input, emb_donor, emb_project, W1, b1, W2, b2):
    didx = input[:, 0].astype(jnp.int32)
    pidx = input[:, 1].astype(jnp.int32)
    rows_d, rows_p = _sc_gather(didx, pidx, emb_donor, emb_project)
    w1t = W1.T
    y = _mlp(rows_d, rows_p, w1t[:_D], w1t[_D:],
             b1.reshape(1, _LIN), W2.T, b2.reshape(1, 1))
    return y
